# baseline (speedup 1.0000x reference)
import functools

import jax
import jax.numpy as jnp
from jax import lax
from jax.experimental import pallas as pl
from jax.experimental.pallas import tpu as pltpu
from jax.experimental.pallas import tpu_sc as plsc

NUM_CODEBOOKS = 32
VOCAB_SIZE = 2051
HIDDEN_SIZE = 2048
SEQ = 2048

_info = plsc.get_sparse_core_info()
_NC, _NS, _L = _info.num_cores, _info.num_subcores, _info.num_lanes
_NW = _NC * _NS
_TOK_PER_W = SEQ // _NW
_HALF = NUM_CODEBOOKS // 2
_OUT_TOK = 16
_VPR = HIDDEN_SIZE // 16


def _sum_half(stage, outchunk, tslot, accumulate):
    @plsc.parallel_loop(0, _VPR, unroll=8)
    def jbody(j):
        sl = pl.ds(j * 16, 16)
        s = stage[0, sl]
        for c in range(1, _HALF):
            s = s + stage[c, sl]
        if accumulate:
            s = s + outchunk[tslot, sl]
        outchunk[tslot, sl] = s


def _body(table_hbm, ids_hbm, out_hbm, idx_v, stage_a, stage_b, outchunk,
          sem_a, sem_b, sem_out):
    wid = lax.axis_index("s") * _NC + lax.axis_index("c")
    wbase = wid * _TOK_PER_W

    pltpu.sync_copy(ids_hbm.at[pl.ds(wbase * NUM_CODEBOOKS, _TOK_PER_W * NUM_CODEBOOKS)], idx_v)
    off_lo = lax.iota(jnp.int32, 16) * VOCAB_SIZE
    off_hi = off_lo + 16 * VOCAB_SIZE

    @plsc.parallel_loop(0, _TOK_PER_W, unroll=4)
    def obody(k):
        lo = pl.ds(k * NUM_CODEBOOKS, 16)
        hi = pl.ds(k * NUM_CODEBOOKS + 16, 16)
        idx_v[lo] = idx_v[lo] + off_lo
        idx_v[hi] = idx_v[hi] + off_hi

    def gather_a(p):
        return pltpu.async_copy(
            table_hbm.at[idx_v.at[pl.ds(p * NUM_CODEBOOKS, _HALF)]], stage_a, sem_a)

    def gather_b(p):
        return pltpu.async_copy(
            table_hbm.at[idx_v.at[pl.ds(p * NUM_CODEBOOKS + _HALF, _HALF)]], stage_b, sem_b)

    gather_a(0)
    gather_b(0)

    def tbody(p, carry):
        tslot = lax.rem(p, _OUT_TOK)

        @pl.when(jnp.logical_and(tslot == 0, p > 0))
        def _():
            pltpu.make_async_copy(
                outchunk, out_hbm.at[pl.ds(0, _OUT_TOK)], sem_out
            ).wait()

        pltpu.make_async_copy(
            table_hbm.at[idx_v.at[pl.ds(p * NUM_CODEBOOKS, _HALF)]], stage_a, sem_a
        ).wait()
        _sum_half(stage_a, outchunk, tslot, accumulate=False)

        @pl.when(p < _TOK_PER_W - 1)
        def _():
            gather_a(p + 1)

        pltpu.make_async_copy(
            table_hbm.at[idx_v.at[pl.ds(p * NUM_CODEBOOKS + _HALF, _HALF)]], stage_b, sem_b
        ).wait()
        _sum_half(stage_b, outchunk, tslot, accumulate=True)

        @pl.when(p < _TOK_PER_W - 1)
        def _():
            gather_b(p + 1)

        @pl.when(tslot == _OUT_TOK - 1)
        def _():
            row0 = pl.multiple_of(wbase + p - (_OUT_TOK - 1), _OUT_TOK)
            pltpu.async_copy(outchunk, out_hbm.at[pl.ds(row0, _OUT_TOK)], sem_out)

        return carry

    lax.fori_loop(0, _TOK_PER_W, tbody, 0)

    pltpu.make_async_copy(outchunk, out_hbm.at[pl.ds(0, _OUT_TOK)], sem_out).wait()


@functools.partial(jax.jit, static_argnames=())
def kernel(input_ids, embed_audio_tokens_weight):
    b, s, ncb = input_ids.shape
    ids_flat = input_ids.reshape(s * ncb).astype(jnp.int32)
    mesh = plsc.VectorSubcoreMesh(core_axis_name="c", subcore_axis_name="s")
    run = pl.kernel(
        _body,
        out_type=jax.ShapeDtypeStruct((SEQ, HIDDEN_SIZE), jnp.float32),
        mesh=mesh,
        scratch_types=[
            pltpu.VMEM((_TOK_PER_W * NUM_CODEBOOKS,), jnp.int32),
            pltpu.VMEM((_HALF, HIDDEN_SIZE), jnp.float32),
            pltpu.VMEM((_HALF, HIDDEN_SIZE), jnp.float32),
            pltpu.VMEM((_OUT_TOK, HIDDEN_SIZE), jnp.float32),
            pltpu.SemaphoreType.DMA,
            pltpu.SemaphoreType.DMA,
            pltpu.SemaphoreType.DMA,
        ],
    )
    out = run(embed_audio_tokens_weight, ids_flat)
    return out.reshape(b, s, HIDDEN_SIZE)


# speedup vs baseline: 1.0079x; 1.0079x over previous
import functools

import jax
import jax.numpy as jnp
from jax import lax
from jax.experimental import pallas as pl
from jax.experimental.pallas import tpu as pltpu
from jax.experimental.pallas import tpu_sc as plsc

NUM_CODEBOOKS = 32
VOCAB_SIZE = 2051
HIDDEN_SIZE = 2048
SEQ = 2048

_info = plsc.get_sparse_core_info()
_NC, _NS, _L = _info.num_cores, _info.num_subcores, _info.num_lanes
_NW = _NC * _NS
_TOK_PER_W = SEQ // _NW
_HALF = NUM_CODEBOOKS // 2
_OUT_TOK = 16
_VPR = HIDDEN_SIZE // 16


def _sum_half(stage, outchunk, tslot, accumulate):
    @plsc.parallel_loop(0, _VPR, unroll=4)
    def jbody(j):
        sl = pl.ds(j * 16, 16)
        s = stage[0, sl]
        for c in range(1, _HALF):
            s = s + stage[c, sl]
        if accumulate:
            s = s + outchunk[tslot, sl]
        outchunk[tslot, sl] = s


def _body(table_hbm, ids_hbm, out_hbm, idx_v, stage_a, stage_b, outchunk,
          sem_a, sem_b, sem_out):
    wid = lax.axis_index("s") * _NC + lax.axis_index("c")
    wbase = wid * _TOK_PER_W

    pltpu.sync_copy(ids_hbm.at[pl.ds(wbase * NUM_CODEBOOKS, _TOK_PER_W * NUM_CODEBOOKS)], idx_v)
    off_lo = lax.iota(jnp.int32, 16) * VOCAB_SIZE
    off_hi = off_lo + 16 * VOCAB_SIZE

    @plsc.parallel_loop(0, _TOK_PER_W, unroll=4)
    def obody(k):
        lo = pl.ds(k * NUM_CODEBOOKS, 16)
        hi = pl.ds(k * NUM_CODEBOOKS + 16, 16)
        idx_v[lo] = idx_v[lo] + off_lo
        idx_v[hi] = idx_v[hi] + off_hi

    def gather_a(p):
        return pltpu.async_copy(
            table_hbm.at[idx_v.at[pl.ds(p * NUM_CODEBOOKS, _HALF)]], stage_a, sem_a)

    def gather_b(p):
        return pltpu.async_copy(
            table_hbm.at[idx_v.at[pl.ds(p * NUM_CODEBOOKS + _HALF, _HALF)]], stage_b, sem_b)

    gather_a(0)
    gather_b(0)

    def tbody(p, carry):
        tslot = lax.rem(p, _OUT_TOK)

        @pl.when(jnp.logical_and(tslot == 0, p > 0))
        def _():
            pltpu.make_async_copy(
                outchunk, out_hbm.at[pl.ds(0, _OUT_TOK)], sem_out
            ).wait()

        pltpu.make_async_copy(
            table_hbm.at[idx_v.at[pl.ds(p * NUM_CODEBOOKS, _HALF)]], stage_a, sem_a
        ).wait()
        _sum_half(stage_a, outchunk, tslot, accumulate=False)

        @pl.when(p < _TOK_PER_W - 1)
        def _():
            gather_a(p + 1)

        pltpu.make_async_copy(
            table_hbm.at[idx_v.at[pl.ds(p * NUM_CODEBOOKS + _HALF, _HALF)]], stage_b, sem_b
        ).wait()
        _sum_half(stage_b, outchunk, tslot, accumulate=True)

        @pl.when(p < _TOK_PER_W - 1)
        def _():
            gather_b(p + 1)

        @pl.when(tslot == _OUT_TOK - 1)
        def _():
            row0 = pl.multiple_of(wbase + p - (_OUT_TOK - 1), _OUT_TOK)
            pltpu.async_copy(outchunk, out_hbm.at[pl.ds(row0, _OUT_TOK)], sem_out)

        return carry

    lax.fori_loop(0, _TOK_PER_W, tbody, 0)

    pltpu.make_async_copy(outchunk, out_hbm.at[pl.ds(0, _OUT_TOK)], sem_out).wait()


@functools.partial(jax.jit, static_argnames=())
def kernel(input_ids, embed_audio_tokens_weight):
    b, s, ncb = input_ids.shape
    ids_flat = input_ids.reshape(s * ncb).astype(jnp.int32)
    mesh = plsc.VectorSubcoreMesh(core_axis_name="c", subcore_axis_name="s")
    run = pl.kernel(
        _body,
        out_type=jax.ShapeDtypeStruct((SEQ, HIDDEN_SIZE), jnp.float32),
        mesh=mesh,
        scratch_types=[
            pltpu.VMEM((_TOK_PER_W * NUM_CODEBOOKS,), jnp.int32),
            pltpu.VMEM((_HALF, HIDDEN_SIZE), jnp.float32),
            pltpu.VMEM((_HALF, HIDDEN_SIZE), jnp.float32),
            pltpu.VMEM((_OUT_TOK, HIDDEN_SIZE), jnp.float32),
            pltpu.SemaphoreType.DMA,
            pltpu.SemaphoreType.DMA,
            pltpu.SemaphoreType.DMA,
        ],
    )
    out = run(embed_audio_tokens_weight, ids_flat)
    return out.reshape(b, s, HIDDEN_SIZE)


# speedup vs baseline: 1.2115x; 1.2020x over previous
import functools

import jax
import jax.numpy as jnp
from jax import lax
from jax.experimental import pallas as pl
from jax.experimental.pallas import tpu as pltpu
from jax.experimental.pallas import tpu_sc as plsc

NUM_CODEBOOKS = 32
VOCAB_SIZE = 2051
HIDDEN_SIZE = 2048
SEQ = 2048

_info = plsc.get_sparse_core_info()
_NC, _NS, _L = _info.num_cores, _info.num_subcores, _info.num_lanes
_NW = _NC * _NS
_TOK_PER_W = SEQ // _NW
_NQ = 4
_QROWS = NUM_CODEBOOKS // _NQ
_OUT_TOK = 16
_VPR = HIDDEN_SIZE // 16


def _sum_quarter(stage, outchunk, tslot, accumulate):
    @plsc.parallel_loop(0, _VPR, unroll=4)
    def jbody(j):
        sl = pl.ds(j * 16, 16)
        s = stage[0, sl]
        for c in range(1, _QROWS):
            s = s + stage[c, sl]
        if accumulate:
            s = s + outchunk[tslot, sl]
        outchunk[tslot, sl] = s


def _body(table_hbm, ids_hbm, out_hbm, idx_v, st0, st1, st2, st3, outchunk,
          sem0, sem1, sem2, sem3, sem_out):
    stages = (st0, st1, st2, st3)
    sems = (sem0, sem1, sem2, sem3)
    wid = lax.axis_index("s") * _NC + lax.axis_index("c")
    wbase = wid * _TOK_PER_W

    pltpu.sync_copy(
        ids_hbm.at[pl.ds(wbase * NUM_CODEBOOKS, _TOK_PER_W * NUM_CODEBOOKS)], idx_v)
    off_lo = lax.iota(jnp.int32, 16) * VOCAB_SIZE
    off_hi = off_lo + 16 * VOCAB_SIZE

    @plsc.parallel_loop(0, _TOK_PER_W, unroll=4)
    def obody(k):
        lo = pl.ds(k * NUM_CODEBOOKS, 16)
        hi = pl.ds(k * NUM_CODEBOOKS + 16, 16)
        idx_v[lo] = idx_v[lo] + off_lo
        idx_v[hi] = idx_v[hi] + off_hi

    def gather(p, q):
        return pltpu.async_copy(
            table_hbm.at[idx_v.at[pl.ds(p * NUM_CODEBOOKS + q * _QROWS, _QROWS)]],
            stages[q], sems[q])

    for q in range(_NQ):
        gather(0, q)

    def tbody(p, carry):
        tslot = lax.rem(p, _OUT_TOK)

        @pl.when(jnp.logical_and(tslot == 0, p > 0))
        def _():
            pltpu.make_async_copy(
                outchunk, out_hbm.at[pl.ds(0, _OUT_TOK)], sem_out
            ).wait()

        for q in range(_NQ):
            pltpu.make_async_copy(
                table_hbm.at[idx_v.at[pl.ds(p * NUM_CODEBOOKS + q * _QROWS, _QROWS)]],
                stages[q], sems[q],
            ).wait()
            _sum_quarter(stages[q], outchunk, tslot, accumulate=q > 0)

            @pl.when(p < _TOK_PER_W - 1)
            def _():
                gather(p + 1, q)

        @pl.when(tslot == _OUT_TOK - 1)
        def _():
            row0 = pl.multiple_of(wbase + p - (_OUT_TOK - 1), _OUT_TOK)
            pltpu.async_copy(outchunk, out_hbm.at[pl.ds(row0, _OUT_TOK)], sem_out)

        return carry

    lax.fori_loop(0, _TOK_PER_W, tbody, 0)

    pltpu.make_async_copy(outchunk, out_hbm.at[pl.ds(0, _OUT_TOK)], sem_out).wait()


@functools.partial(jax.jit, static_argnames=())
def kernel(input_ids, embed_audio_tokens_weight):
    b, s, ncb = input_ids.shape
    ids_flat = input_ids.reshape(s * ncb).astype(jnp.int32)
    mesh = plsc.VectorSubcoreMesh(core_axis_name="c", subcore_axis_name="s")
    run = pl.kernel(
        _body,
        out_type=jax.ShapeDtypeStruct((SEQ, HIDDEN_SIZE), jnp.float32),
        mesh=mesh,
        scratch_types=[
            pltpu.VMEM((_TOK_PER_W * NUM_CODEBOOKS,), jnp.int32),
            pltpu.VMEM((_QROWS, HIDDEN_SIZE), jnp.float32),
            pltpu.VMEM((_QROWS, HIDDEN_SIZE), jnp.float32),
            pltpu.VMEM((_QROWS, HIDDEN_SIZE), jnp.float32),
            pltpu.VMEM((_QROWS, HIDDEN_SIZE), jnp.float32),
            pltpu.VMEM((_OUT_TOK, HIDDEN_SIZE), jnp.float32),
            pltpu.SemaphoreType.DMA,
            pltpu.SemaphoreType.DMA,
            pltpu.SemaphoreType.DMA,
            pltpu.SemaphoreType.DMA,
            pltpu.SemaphoreType.DMA,
        ],
    )
    out = run(embed_audio_tokens_weight, ids_flat)
    return out.reshape(b, s, HIDDEN_SIZE)
